# Initial kernel scaffold; baseline (speedup 1.0000x reference)
import jax
import jax.numpy as jnp
from jax.experimental import pallas as pl


def kernel(node_features, edge_features, edge_index, Wn, bn, We, be, Wm1, bm1, Wm2, bm2, Wu1, bu1, Wu2, bu2):
    raise NotImplementedError("write your pallas kernel here")


# speedup vs baseline: 1.3600x; 1.3600x over previous
import jax
import jax.numpy as jnp
from jax import lax
from jax.experimental import pallas as pl
from jax.experimental.pallas import tpu as pltpu
from jax.experimental.pallas import tpu_sc as plsc

_N = 10000
_E = 320000
_ND = 128
_ED = 16
_H = 64
_LAYERS = 3

_NPAD = 10112
_RPT = _NPAD // 16
_NW = 32
_EPW = 10240
_EPAD = _NW * _EPW
_SUB = 128

_NBLK = 1000
_EBLK = 4096

_F32 = jnp.float32


def _dot(a, b):
    return jnp.dot(a, b, preferred_element_type=_F32,
                   precision=lax.Precision.HIGHEST)


def _node_pre_body(nf, wn, bn, wa, wb, x_o, xab_o):
    x = _dot(nf[...], wn[...]) + bn[...]
    x_o[...] = x
    xab_o[:, :_H] = _dot(x, wa[...])
    xab_o[:, _H:] = _dot(x, wb[...])


_node_pre = pl.pallas_call(
    _node_pre_body,
    grid=(_N // _NBLK,),
    in_specs=[
        pl.BlockSpec((_NBLK, _ND), lambda i: (i, 0)),
        pl.BlockSpec((_ND, _H), lambda i: (0, 0)),
        pl.BlockSpec((1, _H), lambda i: (0, 0)),
        pl.BlockSpec((_H, _H), lambda i: (0, 0)),
        pl.BlockSpec((_H, _H), lambda i: (0, 0)),
    ],
    out_specs=[pl.BlockSpec((_NBLK, _H), lambda i: (i, 0)),
               pl.BlockSpec((_NBLK, 2 * _H), lambda i: (i, 0))],
    out_shape=[jax.ShapeDtypeStruct((_N, _H), _F32),
               jax.ShapeDtypeStruct((_N, 2 * _H), _F32)],
)


def _edge_pre_body(ef2, we, be, wc, bm1, ec_o):
    w2 = _dot(we[...], wc[...])
    b2 = _dot(be[...], wc[...]) + bm1[...]
    ec_o[:, :_H] = _dot(ef2[:, :_ED], w2) + b2
    ec_o[:, _H:] = _dot(ef2[:, _ED:], w2) + b2


_EBLK2 = 2048

_edge_pre = pl.pallas_call(
    _edge_pre_body,
    grid=(_EPAD // 2 // _EBLK2,),
    in_specs=[
        pl.BlockSpec((_EBLK2, 2 * _ED), lambda i: (i, 0)),
        pl.BlockSpec((_ED, _H), lambda i: (0, 0)),
        pl.BlockSpec((1, _H), lambda i: (0, 0)),
        pl.BlockSpec((_H, _H), lambda i: (0, 0)),
        pl.BlockSpec((1, _H), lambda i: (0, 0)),
    ],
    out_specs=pl.BlockSpec((_EBLK2, 2 * _H), lambda i: (i, 0)),
    out_shape=jax.ShapeDtypeStruct((_EPAD // 2, 2 * _H), _F32),
)


def _make_update(with_ab):
    def body(x, a0, a1, d0, d1, wm2, bm2, wu1a, wu1b, bu1, wu2, bu2, wa, wb,
             *outs):
        agg = (a0[0] + a1[0])[:, :_H]
        deg = (d0[0] + d1[0])[:, :1]
        aggregated = _dot(agg, wm2[...]) + deg * bm2[...]
        pre = _dot(x[...], wu1a[...]) + _dot(aggregated, wu1b[...]) + bu1[...]
        xn = _dot(jnp.maximum(pre, 0.0), wu2[...]) + bu2[...] + x[...]
        outs[0][...] = xn
        if with_ab:
            outs[1][:, :_H] = _dot(xn, wa[...])
            outs[1][:, _H:] = _dot(xn, wb[...])

    return pl.pallas_call(
        body,
        grid=(_N // _NBLK,),
        in_specs=[
            pl.BlockSpec((_NBLK, _H), lambda i: (i, 0)),
            pl.BlockSpec((1, _NBLK, 2 * _H), lambda i: (0, i, 0)),
            pl.BlockSpec((1, _NBLK, 2 * _H), lambda i: (1, i, 0)),
            pl.BlockSpec((1, _NBLK, 2 * _H), lambda i: (0, i, 0)),
            pl.BlockSpec((1, _NBLK, 2 * _H), lambda i: (1, i, 0)),
            pl.BlockSpec((_H, _H), lambda i: (0, 0)),
            pl.BlockSpec((1, _H), lambda i: (0, 0)),
            pl.BlockSpec((_H, _H), lambda i: (0, 0)),
            pl.BlockSpec((_H, _H), lambda i: (0, 0)),
            pl.BlockSpec((1, _H), lambda i: (0, 0)),
            pl.BlockSpec((_H, _H), lambda i: (0, 0)),
            pl.BlockSpec((1, _H), lambda i: (0, 0)),
            pl.BlockSpec((_H, _H), lambda i: (0, 0)),
            pl.BlockSpec((_H, _H), lambda i: (0, 0)),
        ],
        out_specs=([pl.BlockSpec((_NBLK, _H), lambda i: (i, 0)),
                    pl.BlockSpec((_NBLK, 2 * _H), lambda i: (i, 0))]
                   if with_ab else
                   [pl.BlockSpec((_NBLK, _H), lambda i: (i, 0))]),
        out_shape=([jax.ShapeDtypeStruct((_N, _H), _F32),
                    jax.ShapeDtypeStruct((_N, 2 * _H), _F32)]
                   if with_ab else
                   [jax.ShapeDtypeStruct((_N, _H), _F32)]),
    )


_update_mid = _make_update(True)
_update_last = _make_update(False)


_CPW = _EPW // _SUB


def _make_sc_edge():
    mesh = plsc.VectorSubcoreMesh(core_axis_name="c", subcore_axis_name="s")
    out_type = [jax.ShapeDtypeStruct((2, _NPAD, 2 * _H), _F32)]
    scratch = [
        pltpu.VMEM((8, _SUB), jnp.int32),
        pltpu.VMEM((8, _SUB), jnp.int32),
        pltpu.VMEM((_SUB,), jnp.int32),
        pltpu.VMEM((_SUB,), jnp.int32),
        pltpu.VMEM((_SUB, 2 * _H), _F32),
        pltpu.VMEM((_SUB, 2 * _H), _F32),
        pltpu.VMEM((_SUB // 2, 2 * _H), _F32),
        pltpu.VMEM_SHARED((_NPAD, 2 * _H), _F32),
        pltpu.SemaphoreType.DMA,
        pltpu.SemaphoreType.DMA,
    ]

    def body(src2, dst2, xab_h, ec2_h, agg_o,
             idx_s8, idx_d8, idx1_s, idx1_d, buf_a, buf_b, buf_c, agg_sh,
             sem_a, sem_b):
        cid = lax.axis_index("c")
        sid = lax.axis_index("s")
        wid = sid * 2 + cid
        zero16 = jnp.zeros((16,), _F32)

        def zrow(r, carry):
            for c in range(2 * _H // 16):
                buf_a[r, pl.ds(c * 16, 16)] = zero16
            return carry

        lax.fori_loop(0, _SUB, zrow, 0)
        base = pl.multiple_of(sid * _RPT, 8)
        off = 0
        while off < _RPT:
            nrows = min(_SUB, _RPT - off)
            pltpu.sync_copy(buf_a.at[pl.ds(0, nrows)],
                            agg_sh.at[pl.ds(base + off, nrows)])
            off += nrows
        plsc.subcore_barrier()

        def super_body(gg, carry):
            irow = pl.multiple_of(wid * _CPW + gg * 8, 8)
            pltpu.sync_copy(src2.at[pl.ds(irow, 8)], idx_s8)
            pltpu.sync_copy(dst2.at[pl.ds(irow, 8)], idx_d8)

            def sub_body(j, carry1):
                for c in range(_SUB // 16):
                    s = pl.ds(c * 16, 16)
                    idx1_s[s] = idx_s8[j, s]
                    idx1_d[s] = idx_d8[j, s]
                cp_a = pltpu.async_copy(xab_h.at[idx1_s], buf_a, sem_a)
                cp_b = pltpu.async_copy(xab_h.at[idx1_d], buf_b, sem_b)
                g = gg * 8 + j
                cbase = pl.multiple_of(
                    (wid * _EPW) // 2 + g * (_SUB // 2), 8)
                pltpu.sync_copy(ec2_h.at[pl.ds(cbase, _SUB // 2)], buf_c)
                cp_a.wait()
                cp_b.wait()

                def crow(rp, carry2):
                    r0 = 2 * rp
                    r1 = r0 + 1
                    for c in range(_H // 16):
                        s = pl.ds(c * 16, 16)
                        s2 = pl.ds(_H + c * 16, 16)
                        buf_a[r0, s] = jnp.maximum(
                            buf_a[r0, s] + buf_b[r0, s2] + buf_c[rp, s], 0.0)
                        buf_a[r1, s] = jnp.maximum(
                            buf_a[r1, s] + buf_b[r1, s2] + buf_c[rp, s2], 0.0)
                    return carry2

                lax.fori_loop(0, _SUB // 2, crow, 0)
                pltpu.sync_copy(buf_a, agg_sh.at[idx1_d], add=True)
                return carry1

            lax.fori_loop(0, 8, sub_body, 0)
            return carry

        lax.fori_loop(0, _CPW // 8, super_body, 0)
        plsc.subcore_barrier()
        pltpu.sync_copy(agg_sh.at[pl.ds(base, _RPT)],
                        agg_o.at[cid, pl.ds(base, _RPT)])

    return pl.kernel(body, out_type=out_type, mesh=mesh,
                     scratch_types=scratch)


_sc_edge = _make_sc_edge()


def _make_sc_deg():
    mesh = plsc.VectorSubcoreMesh(core_axis_name="c", subcore_axis_name="s")
    out_type = [jax.ShapeDtypeStruct((2, _NPAD, 2 * _H), _F32)]
    scratch = [
        pltpu.VMEM((8, _SUB), jnp.int32),
        pltpu.VMEM((_SUB,), jnp.int32),
        pltpu.VMEM((_SUB, 2 * _H), _F32),
        pltpu.VMEM_SHARED((_NPAD, 2 * _H), _F32),
    ]

    def body(dst2, deg_o, idx_d8, idx1_d, buf, deg_sh):
        cid = lax.axis_index("c")
        sid = lax.axis_index("s")
        wid = sid * 2 + cid
        zero16 = jnp.zeros((16,), _F32)
        one16 = jnp.ones((16,), _F32)

        def zrow(r, carry):
            for c in range(2 * _H // 16):
                buf[r, pl.ds(c * 16, 16)] = zero16
            return carry

        lax.fori_loop(0, _SUB, zrow, 0)
        base = pl.multiple_of(sid * _RPT, 8)
        off = 0
        while off < _RPT:
            nrows = min(_SUB, _RPT - off)
            pltpu.sync_copy(buf.at[pl.ds(0, nrows)],
                            deg_sh.at[pl.ds(base + off, nrows)])
            off += nrows

        def orow(r, carry):
            for c in range(2 * _H // 16):
                buf[r, pl.ds(c * 16, 16)] = one16
            return carry

        lax.fori_loop(0, _SUB, orow, 0)
        plsc.subcore_barrier()

        def super_body(gg, carry):
            irow = pl.multiple_of(wid * _CPW + gg * 8, 8)
            pltpu.sync_copy(dst2.at[pl.ds(irow, 8)], idx_d8)

            def sub_body(j, carry1):
                for c in range(_SUB // 16):
                    s = pl.ds(c * 16, 16)
                    idx1_d[s] = idx_d8[j, s]
                pltpu.sync_copy(buf, deg_sh.at[idx1_d], add=True)
                return carry1

            lax.fori_loop(0, 8, sub_body, 0)
            return carry

        lax.fori_loop(0, _CPW // 8, super_body, 0)
        plsc.subcore_barrier()
        pltpu.sync_copy(deg_sh.at[pl.ds(base, _RPT)],
                        deg_o.at[cid, pl.ds(base, _RPT)])

    return pl.kernel(body, out_type=out_type, mesh=mesh,
                     scratch_types=scratch)


_sc_deg = _make_sc_deg()


def kernel(node_features, edge_features, edge_index,
           Wn, bn, We, be, Wm1, bm1, Wm2, bm2, Wu1, bu1, Wu2, bu2):
    bn2 = bn.reshape(1, _H)
    be2 = be.reshape(1, _H)
    bm1_2 = bm1.reshape(1, _H)
    bm2_2 = bm2.reshape(1, _H)
    bu1_2 = bu1.reshape(1, _H)
    bu2_2 = bu2.reshape(1, _H)
    wm1a = Wm1[:_H]
    wm1b = Wm1[_H:2 * _H]
    wm1e = Wm1[2 * _H:]
    wu1a = Wu1[:_H]
    wu1b = Wu1[_H:]

    x, xab = _node_pre(node_features, Wn, bn2, wm1a, wm1b)
    ef_pad = jnp.pad(edge_features, ((0, _EPAD - _E), (0, 0)))
    ef2 = ef_pad.reshape(_EPAD // 2, 2 * _ED)
    ec = _edge_pre(ef2, We, be2, wm1e, bm1_2)

    pad_idx = jnp.full((_EPAD - _E,), _N, dtype=jnp.int32)
    src2 = jnp.concatenate([edge_index[0], pad_idx]).reshape(
        _EPAD // _SUB, _SUB)
    dst2 = jnp.concatenate([edge_index[1], pad_idx]).reshape(
        _EPAD // _SUB, _SUB)

    (deg,) = _sc_deg(dst2)
    for layer in range(_LAYERS):
        xab_p = jnp.pad(xab, ((0, _NPAD - _N), (0, 0)))
        (agg,) = _sc_edge(src2, dst2, xab_p, ec)
        upd_args = (x, agg, agg, deg, deg, Wm2, bm2_2, wu1a, wu1b, bu1_2,
                    Wu2, bu2_2, wm1a, wm1b)
        if layer < _LAYERS - 1:
            x, xab = _update_mid(*upd_args)
        else:
            (x,) = _update_last(*upd_args)
    return x


# speedup vs baseline: 1.6317x; 1.1997x over previous
import jax
import jax.numpy as jnp
from jax import lax
from jax.experimental import pallas as pl
from jax.experimental.pallas import tpu as pltpu
from jax.experimental.pallas import tpu_sc as plsc

_N = 10000
_E = 320000
_ND = 128
_ED = 16
_H = 64
_LAYERS = 3

_NPAD = 10112
_RPT = _NPAD // 16
_NW = 32
_EPW = 10240
_EPAD = _NW * _EPW
_SUB = 128

_NBLK = 1000
_EBLK = 4096

_F32 = jnp.float32


def _dot(a, b):
    return jnp.dot(a, b, preferred_element_type=_F32,
                   precision=lax.Precision.HIGHEST)


def _node_pre_body(nf, wn, bn, wa, wb, x_o, xab_o):
    x = _dot(nf[...], wn[...]) + bn[...]
    x_o[...] = x
    xab_o[:, :_H] = _dot(x, wa[...])
    xab_o[:, _H:] = _dot(x, wb[...])


_node_pre = pl.pallas_call(
    _node_pre_body,
    grid=(_N // _NBLK,),
    in_specs=[
        pl.BlockSpec((_NBLK, _ND), lambda i: (i, 0)),
        pl.BlockSpec((_ND, _H), lambda i: (0, 0)),
        pl.BlockSpec((1, _H), lambda i: (0, 0)),
        pl.BlockSpec((_H, _H), lambda i: (0, 0)),
        pl.BlockSpec((_H, _H), lambda i: (0, 0)),
    ],
    out_specs=[pl.BlockSpec((_NBLK, _H), lambda i: (i, 0)),
               pl.BlockSpec((_NBLK, 2 * _H), lambda i: (i, 0))],
    out_shape=[jax.ShapeDtypeStruct((_N, _H), _F32),
               jax.ShapeDtypeStruct((_N, 2 * _H), _F32)],
)


def _edge_pre_body(ef2, we, be, wc, bm1, ec_o):
    w2 = _dot(we[...], wc[...])
    b2 = _dot(be[...], wc[...]) + bm1[...]
    ec_o[:, :_H] = _dot(ef2[:, :_ED], w2) + b2
    ec_o[:, _H:] = _dot(ef2[:, _ED:], w2) + b2


_EBLK2 = 2048

_edge_pre = pl.pallas_call(
    _edge_pre_body,
    grid=(_EPAD // 2 // _EBLK2,),
    in_specs=[
        pl.BlockSpec((_EBLK2, 2 * _ED), lambda i: (i, 0)),
        pl.BlockSpec((_ED, _H), lambda i: (0, 0)),
        pl.BlockSpec((1, _H), lambda i: (0, 0)),
        pl.BlockSpec((_H, _H), lambda i: (0, 0)),
        pl.BlockSpec((1, _H), lambda i: (0, 0)),
    ],
    out_specs=pl.BlockSpec((_EBLK2, 2 * _H), lambda i: (i, 0)),
    out_shape=jax.ShapeDtypeStruct((_EPAD // 2, 2 * _H), _F32),
)


def _make_update(with_ab):
    def body(x, a0, a1, d0, d1, wm2, bm2, wu1a, wu1b, bu1, wu2, bu2, wa, wb,
             *outs):
        agg = (a0[0] + a1[0])[:, :_H]
        deg = (d0[0] + d1[0])[:, :1]
        aggregated = _dot(agg, wm2[...]) + deg * bm2[...]
        pre = _dot(x[...], wu1a[...]) + _dot(aggregated, wu1b[...]) + bu1[...]
        xn = _dot(jnp.maximum(pre, 0.0), wu2[...]) + bu2[...] + x[...]
        outs[0][...] = xn
        if with_ab:
            outs[1][:, :_H] = _dot(xn, wa[...])
            outs[1][:, _H:] = _dot(xn, wb[...])

    return pl.pallas_call(
        body,
        grid=(_N // _NBLK,),
        in_specs=[
            pl.BlockSpec((_NBLK, _H), lambda i: (i, 0)),
            pl.BlockSpec((1, _NBLK, 2 * _H), lambda i: (0, i, 0)),
            pl.BlockSpec((1, _NBLK, 2 * _H), lambda i: (1, i, 0)),
            pl.BlockSpec((1, _NBLK, 2 * _H), lambda i: (0, i, 0)),
            pl.BlockSpec((1, _NBLK, 2 * _H), lambda i: (1, i, 0)),
            pl.BlockSpec((_H, _H), lambda i: (0, 0)),
            pl.BlockSpec((1, _H), lambda i: (0, 0)),
            pl.BlockSpec((_H, _H), lambda i: (0, 0)),
            pl.BlockSpec((_H, _H), lambda i: (0, 0)),
            pl.BlockSpec((1, _H), lambda i: (0, 0)),
            pl.BlockSpec((_H, _H), lambda i: (0, 0)),
            pl.BlockSpec((1, _H), lambda i: (0, 0)),
            pl.BlockSpec((_H, _H), lambda i: (0, 0)),
            pl.BlockSpec((_H, _H), lambda i: (0, 0)),
        ],
        out_specs=([pl.BlockSpec((_NBLK, _H), lambda i: (i, 0)),
                    pl.BlockSpec((_NBLK, 2 * _H), lambda i: (i, 0))]
                   if with_ab else
                   [pl.BlockSpec((_NBLK, _H), lambda i: (i, 0))]),
        out_shape=([jax.ShapeDtypeStruct((_N, _H), _F32),
                    jax.ShapeDtypeStruct((_N, 2 * _H), _F32)]
                   if with_ab else
                   [jax.ShapeDtypeStruct((_N, _H), _F32)]),
    )


_update_mid = _make_update(True)
_update_last = _make_update(False)


_CPW = _EPW // _SUB


_CH = 64


def _make_sc_edge():
    mesh = plsc.VectorSubcoreMesh(core_axis_name="c", subcore_axis_name="s")
    out_type = [jax.ShapeDtypeStruct((2, _NPAD, 2 * _H), _F32)]
    scratch = [
        pltpu.VMEM((8, _SUB), jnp.int32),
        pltpu.VMEM((8, _SUB), jnp.int32),
        pltpu.VMEM((_CH,), jnp.int32),
        pltpu.VMEM((_CH,), jnp.int32),
        pltpu.VMEM((_CH,), jnp.int32),
        pltpu.VMEM((_CH,), jnp.int32),
        pltpu.VMEM((_CH, 2 * _H), _F32),
        pltpu.VMEM((_CH, 2 * _H), _F32),
        pltpu.VMEM((_CH, 2 * _H), _F32),
        pltpu.VMEM((_CH, 2 * _H), _F32),
        pltpu.VMEM((_CH // 2, 2 * _H), _F32),
        pltpu.VMEM((_CH // 2, 2 * _H), _F32),
        pltpu.VMEM_SHARED((_NPAD, 2 * _H), _F32),
        pltpu.SemaphoreType.DMA,
        pltpu.SemaphoreType.DMA,
        pltpu.SemaphoreType.DMA,
        pltpu.SemaphoreType.DMA,
        pltpu.SemaphoreType.DMA,
        pltpu.SemaphoreType.DMA,
    ]

    def body(src2, dst2, xab_h, ec2_h, agg_o,
             idx_s8, idx_d8, is0, is1, id0, id1, a0, a1, b0, b1, c0, c1,
             agg_sh, sa0, sa1, sb0, sb1, se0, se1):
        cid = lax.axis_index("c")
        sid = lax.axis_index("s")
        wid = sid * 2 + cid
        i1s = (is0, is1)
        i1d = (id0, id1)
        bfa = (a0, a1)
        bfb = (b0, b1)
        bfc = (c0, c1)
        sma = (sa0, sa1)
        smb = (sb0, sb1)
        sme = (se0, se1)
        zero16 = jnp.zeros((16,), _F32)

        def zrow(r, carry):
            for c in range(2 * _H // 16):
                a0[r, pl.ds(c * 16, 16)] = zero16
            return carry

        lax.fori_loop(0, _CH, zrow, 0)
        base = pl.multiple_of(sid * _RPT, 8)
        off = 0
        while off < _RPT:
            nrows = min(_CH, _RPT - off)
            pltpu.sync_copy(a0.at[pl.ds(0, nrows)],
                            agg_sh.at[pl.ds(base + off, nrows)])
            off += nrows
        plsc.subcore_barrier()

        nchunks = _EPW // _CH
        rows_per_blk = 8

        def load_idx_block(blk):
            irow = pl.multiple_of(wid * _CPW + blk * rows_per_blk, 8)
            pltpu.sync_copy(src2.at[pl.ds(irow, rows_per_blk)], idx_s8)
            pltpu.sync_copy(dst2.at[pl.ds(irow, rows_per_blk)], idx_d8)

        def stage_idx(c, p, col):
            row = (c % 16) // 2
            dst_s = i1s[p]
            dst_d = i1d[p]
            for k in range(_CH // 16):
                s_src = pl.ds(col + k * 16, 16)
                s_dst = pl.ds(k * 16, 16)
                dst_s[s_dst] = idx_s8[row, s_src]
                dst_d[s_dst] = idx_d8[row, s_src]

        def fire(c, p):
            cp1 = pltpu.async_copy(xab_h.at[i1s[p]], bfa[p], sma[p])
            cp2 = pltpu.async_copy(xab_h.at[i1d[p]], bfb[p], smb[p])
            cbase = pl.multiple_of(
                (wid * _EPW) // 2 + c * (_CH // 2), 8)
            cp3 = pltpu.async_copy(ec2_h.at[pl.ds(cbase, _CH // 2)],
                                   bfc[p], sme[p])
            return cp1, cp2, cp3

        def wait_chunk(p):
            pltpu.make_async_copy(xab_h.at[i1s[p]], bfa[p], sma[p]).wait()
            pltpu.make_async_copy(xab_h.at[i1d[p]], bfb[p], smb[p]).wait()
            pltpu.make_async_copy(ec2_h.at[pl.ds(0, _CH // 2)],
                                  bfc[p], sme[p]).wait()

        def compute_scatter(p):
            ba = bfa[p]
            bb = bfb[p]
            bc = bfc[p]

            def crow(rp, carry2):
                r0 = 2 * rp
                r1 = r0 + 1
                for c in range(_H // 16):
                    s = pl.ds(c * 16, 16)
                    s2 = pl.ds(_H + c * 16, 16)
                    ba[r0, s] = jnp.maximum(
                        ba[r0, s] + bb[r0, s2] + bc[rp, s], 0.0)
                    ba[r1, s] = jnp.maximum(
                        ba[r1, s] + bb[r1, s2] + bc[rp, s2], 0.0)
                return carry2

            lax.fori_loop(0, _CH // 2, crow, 0)
            pltpu.sync_copy(ba, agg_sh.at[i1d[p]], add=True)

        load_idx_block(0)
        stage_idx(0, 0, 0)
        fire(0, 0)

        def outer(g2, carry):
            for j in range(2):
                c = 2 * g2 + j
                p = j
                if j == 0:
                    stage_idx(c + 1, 1, 64)
                    fire(c + 1, 1)
                else:
                    @pl.when(c + 1 < nchunks)
                    def _():
                        @pl.when((c + 1) % 16 == 0)
                        def _():
                            load_idx_block((c + 1) // 16)
                        stage_idx(c + 1, 0, 0)
                        fire(c + 1, 0)
                wait_chunk(p)
                compute_scatter(p)
            return carry

        lax.fori_loop(0, nchunks // 2, outer, 0)
        plsc.subcore_barrier()
        pltpu.sync_copy(agg_sh.at[pl.ds(base, _RPT)],
                        agg_o.at[cid, pl.ds(base, _RPT)])

    return pl.kernel(body, out_type=out_type, mesh=mesh,
                     scratch_types=scratch)


_sc_edge = _make_sc_edge()


def _make_sc_deg():
    mesh = plsc.VectorSubcoreMesh(core_axis_name="c", subcore_axis_name="s")
    out_type = [jax.ShapeDtypeStruct((2, _NPAD, 2 * _H), _F32)]
    scratch = [
        pltpu.VMEM((8, _SUB), jnp.int32),
        pltpu.VMEM((_SUB,), jnp.int32),
        pltpu.VMEM((_SUB, 2 * _H), _F32),
        pltpu.VMEM_SHARED((_NPAD, 2 * _H), _F32),
    ]

    def body(dst2, deg_o, idx_d8, idx1_d, buf, deg_sh):
        cid = lax.axis_index("c")
        sid = lax.axis_index("s")
        wid = sid * 2 + cid
        zero16 = jnp.zeros((16,), _F32)
        one16 = jnp.ones((16,), _F32)

        def zrow(r, carry):
            for c in range(2 * _H // 16):
                buf[r, pl.ds(c * 16, 16)] = zero16
            return carry

        lax.fori_loop(0, _SUB, zrow, 0)
        base = pl.multiple_of(sid * _RPT, 8)
        off = 0
        while off < _RPT:
            nrows = min(_SUB, _RPT - off)
            pltpu.sync_copy(buf.at[pl.ds(0, nrows)],
                            deg_sh.at[pl.ds(base + off, nrows)])
            off += nrows

        def orow(r, carry):
            for c in range(2 * _H // 16):
                buf[r, pl.ds(c * 16, 16)] = one16
            return carry

        lax.fori_loop(0, _SUB, orow, 0)
        plsc.subcore_barrier()

        def super_body(gg, carry):
            irow = pl.multiple_of(wid * _CPW + gg * 8, 8)
            pltpu.sync_copy(dst2.at[pl.ds(irow, 8)], idx_d8)

            def sub_body(j, carry1):
                for c in range(_SUB // 16):
                    s = pl.ds(c * 16, 16)
                    idx1_d[s] = idx_d8[j, s]
                pltpu.sync_copy(buf, deg_sh.at[idx1_d], add=True)
                return carry1

            lax.fori_loop(0, 8, sub_body, 0)
            return carry

        lax.fori_loop(0, _CPW // 8, super_body, 0)
        plsc.subcore_barrier()
        pltpu.sync_copy(deg_sh.at[pl.ds(base, _RPT)],
                        deg_o.at[cid, pl.ds(base, _RPT)])

    return pl.kernel(body, out_type=out_type, mesh=mesh,
                     scratch_types=scratch)


_sc_deg = _make_sc_deg()


def kernel(node_features, edge_features, edge_index,
           Wn, bn, We, be, Wm1, bm1, Wm2, bm2, Wu1, bu1, Wu2, bu2):
    bn2 = bn.reshape(1, _H)
    be2 = be.reshape(1, _H)
    bm1_2 = bm1.reshape(1, _H)
    bm2_2 = bm2.reshape(1, _H)
    bu1_2 = bu1.reshape(1, _H)
    bu2_2 = bu2.reshape(1, _H)
    wm1a = Wm1[:_H]
    wm1b = Wm1[_H:2 * _H]
    wm1e = Wm1[2 * _H:]
    wu1a = Wu1[:_H]
    wu1b = Wu1[_H:]

    x, xab = _node_pre(node_features, Wn, bn2, wm1a, wm1b)
    ef_pad = jnp.pad(edge_features, ((0, _EPAD - _E), (0, 0)))
    ef2 = ef_pad.reshape(_EPAD // 2, 2 * _ED)
    ec = _edge_pre(ef2, We, be2, wm1e, bm1_2)

    pad_idx = jnp.full((_EPAD - _E,), _N, dtype=jnp.int32)
    src2 = jnp.concatenate([edge_index[0], pad_idx]).reshape(
        _EPAD // _SUB, _SUB)
    dst2 = jnp.concatenate([edge_index[1], pad_idx]).reshape(
        _EPAD // _SUB, _SUB)

    (deg,) = _sc_deg(dst2)
    for layer in range(_LAYERS):
        xab_p = jnp.pad(xab, ((0, _NPAD - _N), (0, 0)))
        (agg,) = _sc_edge(src2, dst2, xab_p, ec)
        upd_args = (x, agg, agg, deg, deg, Wm2, bm2_2, wu1a, wu1b, bu1_2,
                    Wu2, bu2_2, wm1a, wm1b)
        if layer < _LAYERS - 1:
            x, xab = _update_mid(*upd_args)
        else:
            (x,) = _update_last(*upd_args)
    return x


# speedup vs baseline: 1.7339x; 1.0626x over previous
import jax
import jax.numpy as jnp
from jax import lax
from jax.experimental import pallas as pl
from jax.experimental.pallas import tpu as pltpu
from jax.experimental.pallas import tpu_sc as plsc

_N = 10000
_E = 320000
_ND = 128
_ED = 16
_H = 64
_LAYERS = 3

_NPAD = 10112
_RPT = _NPAD // 16
_NW = 32
_EPW = 10240
_EPAD = _NW * _EPW
_SUB = 128

_NBLK = 1000
_EBLK = 4096

_F32 = jnp.float32


def _dot(a, b):
    return jnp.dot(a, b, preferred_element_type=_F32,
                   precision=lax.Precision.HIGHEST)


def _node_pre_body(nf, wn, bn, wa, wb, x_o, xab_o):
    x = _dot(nf[...], wn[...]) + bn[...]
    x_o[...] = x
    xab_o[:, :_H] = _dot(x, wa[...])
    xab_o[:, _H:] = _dot(x, wb[...])


_node_pre = pl.pallas_call(
    _node_pre_body,
    grid=(_N // _NBLK,),
    in_specs=[
        pl.BlockSpec((_NBLK, _ND), lambda i: (i, 0)),
        pl.BlockSpec((_ND, _H), lambda i: (0, 0)),
        pl.BlockSpec((1, _H), lambda i: (0, 0)),
        pl.BlockSpec((_H, _H), lambda i: (0, 0)),
        pl.BlockSpec((_H, _H), lambda i: (0, 0)),
    ],
    out_specs=[pl.BlockSpec((_NBLK, _H), lambda i: (i, 0)),
               pl.BlockSpec((_NBLK, 2 * _H), lambda i: (i, 0))],
    out_shape=[jax.ShapeDtypeStruct((_N, _H), _F32),
               jax.ShapeDtypeStruct((_N, 2 * _H), _F32)],
)


def _edge_pre_body(ef2, we, be, wc, bm1, ec_o):
    w2 = _dot(we[...], wc[...])
    b2 = _dot(be[...], wc[...]) + bm1[...]
    ec_o[:, :_H] = _dot(ef2[:, :_ED], w2) + b2
    ec_o[:, _H:] = _dot(ef2[:, _ED:], w2) + b2


_EBLK2 = 2048

_edge_pre = pl.pallas_call(
    _edge_pre_body,
    grid=(_EPAD // 2 // _EBLK2,),
    in_specs=[
        pl.BlockSpec((_EBLK2, 2 * _ED), lambda i: (i, 0)),
        pl.BlockSpec((_ED, _H), lambda i: (0, 0)),
        pl.BlockSpec((1, _H), lambda i: (0, 0)),
        pl.BlockSpec((_H, _H), lambda i: (0, 0)),
        pl.BlockSpec((1, _H), lambda i: (0, 0)),
    ],
    out_specs=pl.BlockSpec((_EBLK2, 2 * _H), lambda i: (i, 0)),
    out_shape=jax.ShapeDtypeStruct((_EPAD // 2, 2 * _H), _F32),
)


def _make_update(with_ab):
    def body(x, a0, a1, d0, d1, wm2, bm2, wu1a, wu1b, bu1, wu2, bu2, wa, wb,
             *outs):
        agg = (a0[0] + a1[0])[:, :_H]
        deg = (d0[0] + d1[0])[:, :1]
        aggregated = _dot(agg, wm2[...]) + deg * bm2[...]
        pre = _dot(x[...], wu1a[...]) + _dot(aggregated, wu1b[...]) + bu1[...]
        xn = _dot(jnp.maximum(pre, 0.0), wu2[...]) + bu2[...] + x[...]
        outs[0][...] = xn
        if with_ab:
            outs[1][:, :_H] = _dot(xn, wa[...])
            outs[1][:, _H:] = _dot(xn, wb[...])

    return pl.pallas_call(
        body,
        grid=(_N // _NBLK,),
        in_specs=[
            pl.BlockSpec((_NBLK, _H), lambda i: (i, 0)),
            pl.BlockSpec((1, _NBLK, 2 * _H), lambda i: (0, i, 0)),
            pl.BlockSpec((1, _NBLK, 2 * _H), lambda i: (1, i, 0)),
            pl.BlockSpec((1, _NBLK, 2 * _H), lambda i: (0, i, 0)),
            pl.BlockSpec((1, _NBLK, 2 * _H), lambda i: (1, i, 0)),
            pl.BlockSpec((_H, _H), lambda i: (0, 0)),
            pl.BlockSpec((1, _H), lambda i: (0, 0)),
            pl.BlockSpec((_H, _H), lambda i: (0, 0)),
            pl.BlockSpec((_H, _H), lambda i: (0, 0)),
            pl.BlockSpec((1, _H), lambda i: (0, 0)),
            pl.BlockSpec((_H, _H), lambda i: (0, 0)),
            pl.BlockSpec((1, _H), lambda i: (0, 0)),
            pl.BlockSpec((_H, _H), lambda i: (0, 0)),
            pl.BlockSpec((_H, _H), lambda i: (0, 0)),
        ],
        out_specs=([pl.BlockSpec((_NBLK, _H), lambda i: (i, 0)),
                    pl.BlockSpec((_NBLK, 2 * _H), lambda i: (i, 0))]
                   if with_ab else
                   [pl.BlockSpec((_NBLK, _H), lambda i: (i, 0))]),
        out_shape=([jax.ShapeDtypeStruct((_N, _H), _F32),
                    jax.ShapeDtypeStruct((_N, 2 * _H), _F32)]
                   if with_ab else
                   [jax.ShapeDtypeStruct((_N, _H), _F32)]),
    )


_update_mid = _make_update(True)
_update_last = _make_update(False)


_CPW = _EPW // _SUB


_CH = 64
_R0 = 112
_R1 = 48


def _make_sc_edge():
    mesh = plsc.VectorSubcoreMesh(core_axis_name="c", subcore_axis_name="s")
    out_type = [jax.ShapeDtypeStruct((2, _NPAD, 2 * _H), _F32)]
    scratch = [
        pltpu.VMEM((8, _SUB), jnp.int32),
        pltpu.VMEM((8, _SUB), jnp.int32),
        pltpu.VMEM((_CH,), jnp.int32),
        pltpu.VMEM((_CH,), jnp.int32),
        pltpu.VMEM((_CH,), jnp.int32),
        pltpu.VMEM((_CH,), jnp.int32),
        pltpu.VMEM((_CH, 2 * _H), _F32),
        pltpu.VMEM((_CH, 2 * _H), _F32),
        pltpu.VMEM((_CH, 2 * _H), _F32),
        pltpu.VMEM((_CH, 2 * _H), _F32),
        pltpu.VMEM((_CH // 2, 2 * _H), _F32),
        pltpu.VMEM((_CH // 2, 2 * _H), _F32),
        pltpu.VMEM_SHARED((_NPAD, 2 * _H), _F32),
        pltpu.SemaphoreType.DMA,
        pltpu.SemaphoreType.DMA,
        pltpu.SemaphoreType.DMA,
        pltpu.SemaphoreType.DMA,
        pltpu.SemaphoreType.DMA,
        pltpu.SemaphoreType.DMA,
    ]

    def body(src2, dst2, xab_h, ec2_h, agg_o,
             idx_s8, idx_d8, is0, is1, id0, id1, a0, a1, b0, b1, c0, c1,
             agg_sh, sa0, sa1, sb0, sb1, se0, se1):
        cid = lax.axis_index("c")
        sid = lax.axis_index("s")
        wid = sid * 2 + cid
        i1s = (is0, is1)
        i1d = (id0, id1)
        bfa = (a0, a1)
        bfb = (b0, b1)
        bfc = (c0, c1)
        sma = (sa0, sa1)
        smb = (sb0, sb1)
        sme = (se0, se1)
        zero16 = jnp.zeros((16,), _F32)

        def zrow(r, carry):
            for c in range(2 * _H // 16):
                a0[r, pl.ds(c * 16, 16)] = zero16
            return carry

        lax.fori_loop(0, _CH, zrow, 0)
        base = pl.multiple_of(sid * _RPT, 8)
        off = 0
        while off < _RPT:
            nrows = min(_CH, _RPT - off)
            pltpu.sync_copy(a0.at[pl.ds(0, nrows)],
                            agg_sh.at[pl.ds(base + off, nrows)])
            off += nrows
        plsc.subcore_barrier()

        rowbase = jnp.where(cid == 0, sid * _R0, 16 * _R0 + sid * _R1)
        nchunks = jnp.where(cid == 0, 2 * _R0, 2 * _R1)
        rows_per_blk = 8

        def load_idx_block(blk):
            irow = pl.multiple_of(rowbase + blk * rows_per_blk, 8)
            pltpu.sync_copy(src2.at[pl.ds(irow, rows_per_blk)], idx_s8)
            pltpu.sync_copy(dst2.at[pl.ds(irow, rows_per_blk)], idx_d8)

        def stage_idx(c, p, col):
            row = (c % 16) // 2
            dst_s = i1s[p]
            dst_d = i1d[p]
            for k in range(_CH // 16):
                s_src = pl.ds(col + k * 16, 16)
                s_dst = pl.ds(k * 16, 16)
                dst_s[s_dst] = idx_s8[row, s_src]
                dst_d[s_dst] = idx_d8[row, s_src]

        def fire(c, p):
            cp1 = pltpu.async_copy(xab_h.at[i1s[p]], bfa[p], sma[p])
            cp2 = pltpu.async_copy(xab_h.at[i1d[p]], bfb[p], smb[p])
            cbase = pl.multiple_of(rowbase * 64 + c * (_CH // 2), 8)
            cp3 = pltpu.async_copy(ec2_h.at[pl.ds(cbase, _CH // 2)],
                                   bfc[p], sme[p])
            return cp1, cp2, cp3

        def wait_chunk(p):
            pltpu.make_async_copy(xab_h.at[i1s[p]], bfa[p], sma[p]).wait()
            pltpu.make_async_copy(xab_h.at[i1d[p]], bfb[p], smb[p]).wait()
            pltpu.make_async_copy(ec2_h.at[pl.ds(0, _CH // 2)],
                                  bfc[p], sme[p]).wait()

        def compute_scatter(p):
            ba = bfa[p]
            bb = bfb[p]
            bc = bfc[p]

            def crow(rp, carry2):
                r0 = 2 * rp
                r1 = r0 + 1
                for c in range(_H // 16):
                    s = pl.ds(c * 16, 16)
                    s2 = pl.ds(_H + c * 16, 16)
                    ba[r0, s] = jnp.maximum(
                        ba[r0, s] + bb[r0, s2] + bc[rp, s], 0.0)
                    ba[r1, s] = jnp.maximum(
                        ba[r1, s] + bb[r1, s2] + bc[rp, s2], 0.0)
                return carry2

            lax.fori_loop(0, _CH // 2, crow, 0)
            pltpu.sync_copy(ba, agg_sh.at[i1d[p]], add=True)

        load_idx_block(0)
        stage_idx(0, 0, 0)
        fire(0, 0)

        def outer(g2, carry):
            for j in range(2):
                c = 2 * g2 + j
                p = j
                if j == 0:
                    stage_idx(c + 1, 1, 64)
                    fire(c + 1, 1)
                else:
                    @pl.when(c + 1 < nchunks)
                    def _():
                        @pl.when((c + 1) % 16 == 0)
                        def _():
                            load_idx_block((c + 1) // 16)
                        stage_idx(c + 1, 0, 0)
                        fire(c + 1, 0)
                wait_chunk(p)
                compute_scatter(p)
            return carry

        lax.fori_loop(0, jnp.where(cid == 0, _R0, _R1), outer, 0)
        plsc.subcore_barrier()
        pltpu.sync_copy(agg_sh.at[pl.ds(base, _RPT)],
                        agg_o.at[cid, pl.ds(base, _RPT)])

    return pl.kernel(body, out_type=out_type, mesh=mesh,
                     scratch_types=scratch)


_sc_edge = _make_sc_edge()


def _make_sc_deg():
    mesh = plsc.VectorSubcoreMesh(core_axis_name="c", subcore_axis_name="s")
    out_type = [jax.ShapeDtypeStruct((2, _NPAD, 2 * _H), _F32)]
    scratch = [
        pltpu.VMEM((8, _SUB), jnp.int32),
        pltpu.VMEM((_SUB,), jnp.int32),
        pltpu.VMEM((_SUB, 2 * _H), _F32),
        pltpu.VMEM_SHARED((_NPAD, 2 * _H), _F32),
    ]

    def body(dst2, deg_o, idx_d8, idx1_d, buf, deg_sh):
        cid = lax.axis_index("c")
        sid = lax.axis_index("s")
        wid = sid * 2 + cid
        zero16 = jnp.zeros((16,), _F32)
        one16 = jnp.ones((16,), _F32)

        def zrow(r, carry):
            for c in range(2 * _H // 16):
                buf[r, pl.ds(c * 16, 16)] = zero16
            return carry

        lax.fori_loop(0, _SUB, zrow, 0)
        base = pl.multiple_of(sid * _RPT, 8)
        off = 0
        while off < _RPT:
            nrows = min(_SUB, _RPT - off)
            pltpu.sync_copy(buf.at[pl.ds(0, nrows)],
                            deg_sh.at[pl.ds(base + off, nrows)])
            off += nrows

        def orow(r, carry):
            for c in range(2 * _H // 16):
                buf[r, pl.ds(c * 16, 16)] = one16
            return carry

        lax.fori_loop(0, _SUB, orow, 0)
        plsc.subcore_barrier()

        def super_body(gg, carry):
            irow = pl.multiple_of(wid * _CPW + gg * 8, 8)
            pltpu.sync_copy(dst2.at[pl.ds(irow, 8)], idx_d8)

            def sub_body(j, carry1):
                for c in range(_SUB // 16):
                    s = pl.ds(c * 16, 16)
                    idx1_d[s] = idx_d8[j, s]
                pltpu.sync_copy(buf, deg_sh.at[idx1_d], add=True)
                return carry1

            lax.fori_loop(0, 8, sub_body, 0)
            return carry

        lax.fori_loop(0, _CPW // 8, super_body, 0)
        plsc.subcore_barrier()
        pltpu.sync_copy(deg_sh.at[pl.ds(base, _RPT)],
                        deg_o.at[cid, pl.ds(base, _RPT)])

    return pl.kernel(body, out_type=out_type, mesh=mesh,
                     scratch_types=scratch)


_sc_deg = _make_sc_deg()


def kernel(node_features, edge_features, edge_index,
           Wn, bn, We, be, Wm1, bm1, Wm2, bm2, Wu1, bu1, Wu2, bu2):
    bn2 = bn.reshape(1, _H)
    be2 = be.reshape(1, _H)
    bm1_2 = bm1.reshape(1, _H)
    bm2_2 = bm2.reshape(1, _H)
    bu1_2 = bu1.reshape(1, _H)
    bu2_2 = bu2.reshape(1, _H)
    wm1a = Wm1[:_H]
    wm1b = Wm1[_H:2 * _H]
    wm1e = Wm1[2 * _H:]
    wu1a = Wu1[:_H]
    wu1b = Wu1[_H:]

    x, xab = _node_pre(node_features, Wn, bn2, wm1a, wm1b)
    ef_pad = jnp.pad(edge_features, ((0, _EPAD - _E), (0, 0)))
    ef2 = ef_pad.reshape(_EPAD // 2, 2 * _ED)
    ec = _edge_pre(ef2, We, be2, wm1e, bm1_2)

    pad_idx = jnp.full((_EPAD - _E,), _N, dtype=jnp.int32)
    src2 = jnp.concatenate([edge_index[0], pad_idx]).reshape(
        _EPAD // _SUB, _SUB)
    dst2 = jnp.concatenate([edge_index[1], pad_idx]).reshape(
        _EPAD // _SUB, _SUB)

    (deg,) = _sc_deg(dst2)
    for layer in range(_LAYERS):
        xab_p = jnp.pad(xab, ((0, _NPAD - _N), (0, 0)))
        (agg,) = _sc_edge(src2, dst2, xab_p, ec)
        upd_args = (x, agg, agg, deg, deg, Wm2, bm2_2, wu1a, wu1b, bu1_2,
                    Wu2, bu2_2, wm1a, wm1b)
        if layer < _LAYERS - 1:
            x, xab = _update_mid(*upd_args)
        else:
            (x,) = _update_last(*upd_args)
    return x
